# Initial kernel scaffold; baseline (speedup 1.0000x reference)
import jax
import jax.numpy as jnp
from jax.experimental import pallas as pl


def kernel(mix, W_enc, b_enc, ln_gamma, ln_beta, W_bn, b_bn, W1, b1, W2, b2, prelu, W_f, b_f, W_g, b_g):
    raise NotImplementedError("write your pallas kernel here")


# speedup vs baseline: 1.0151x; 1.0151x over previous
import functools

import jax
import jax.numpy as jnp
from jax.experimental import pallas as pl
from jax.experimental.pallas import tpu as pltpu

LANE = 128
WIN = 16
CONTEXT = 16


def _round_up(x, m):
    return (x + m - 1) // m * m


def _shift_left(x, n):
    if n == 0:
        return x
    pad = jnp.zeros(x.shape[:-1] + (n,), x.dtype)
    return jnp.concatenate([x[:, n:], pad], axis=1)


def _enc_stats_kernel(xa_ref, w_enc_ref, b_enc_ref, o_ref, *, tm, n_win):
    xa = xa_ref[0]
    enc = jnp.dot(xa.astype(jnp.bfloat16), w_enc_ref[...],
                  preferred_element_type=jnp.float32) + b_enc_ref[...]
    j = pl.program_id(1)
    row = jax.lax.broadcasted_iota(jnp.int32, (tm, 1), 0)
    mask = (row + j * tm) < n_win
    encm = jnp.where(mask, enc, 0.0)
    s = jnp.sum(encm, axis=0, keepdims=True)
    sq = jnp.sum(encm * encm, axis=0, keepdims=True)
    o_ref[...] = jnp.concatenate([s, sq], axis=1)[None, None]


def _pallas_enc_stats(xa4, w_enc, b_enc, *, tm, n_win):
    nchan, nw, _ = xa4.shape
    enc = w_enc.shape[1]
    ntile = nw // tm
    kern = functools.partial(_enc_stats_kernel, tm=tm, n_win=n_win)
    return pl.pallas_call(
        kern,
        out_shape=jax.ShapeDtypeStruct((nchan, ntile, 1, 2 * enc), jnp.float32),
        grid_spec=pltpu.PrefetchScalarGridSpec(
            num_scalar_prefetch=0,
            grid=(nchan, ntile),
            in_specs=[
                pl.BlockSpec((1, tm, LANE), lambda c, j: (c, j, 0)),
                pl.BlockSpec((LANE, enc), lambda c, j: (0, 0)),
                pl.BlockSpec((1, enc), lambda c, j: (0, 0)),
            ],
            out_specs=pl.BlockSpec((1, 1, 1, 2 * enc), lambda c, j: (c, j, 0, 0)),
        ),
        compiler_params=pltpu.CompilerParams(
            dimension_semantics=("parallel", "parallel")),
    )(xa4, w_enc, b_enc)


def _fused_kernel(xa_ref, stats_ref, w_enc_ref, b_enc_ref, gamma_ref, beta_ref,
                  w_bn_ref, b_bn_ref, w1_ref, b1_ref, w2_ref, b2_ref,
                  slope_ref, w_fg_ref, b_fg_ref, o_ref,
                  *, n_layer, n_spk, n_lags, win, cw, enc_dim, half):
    xa = xa_ref[0]
    tm = xa.shape[0]

    enc = jnp.dot(xa.astype(jnp.bfloat16), w_enc_ref[...],
                  preferred_element_type=jnp.float32) + b_enc_ref[...]

    num = jnp.zeros_like(xa)
    seg_sq = jnp.zeros_like(xa)
    for t in range(win):
        sh = _shift_left(xa, t)
        r_t = xa[:, cw + t:cw + t + 1]
        num = num + sh * r_t
        seg_sq = seg_sq + sh * sh
    ref = xa[:, cw:cw + win]
    ref_sq = jnp.sum(ref * ref, axis=1, keepdims=True)
    den = jnp.sqrt(seg_sq * ref_sq) + 1e-8
    ncc = num * pl.reciprocal(den, approx=True)

    st = stats_ref[0]
    mean = st[:, 0:1]
    rstd = st[:, 1:2]
    enc_ln = (enc - mean) * rstd * gamma_ref[...] + beta_ref[...]

    pad = LANE - enc_dim - n_lags
    x_in = jnp.concatenate(
        [enc_ln, ncc[:, :n_lags], jnp.zeros((tm, pad), jnp.float32)], axis=1)
    h = jnp.dot(x_in.astype(jnp.bfloat16), w_bn_ref[...],
                preferred_element_type=jnp.float32) + b_bn_ref[...]

    for l in range(n_layer):
        y = jnp.dot(h.astype(jnp.bfloat16), w1_ref[l],
                    preferred_element_type=jnp.float32) + b1_ref[l]
        y = jnp.where(y >= 0.0, y, slope_ref[l] * y)
        h = h + jnp.dot(y.astype(jnp.bfloat16), w2_ref[l],
                        preferred_element_type=jnp.float32) + b2_ref[l]

    fg = jnp.dot(h.astype(jnp.bfloat16), w_fg_ref[...],
                 preferred_element_type=jnp.float32) + b_fg_ref[...]
    filt = jnp.tanh(fg[:, :half]) * jax.nn.sigmoid(fg[:, half:])

    acc0 = jnp.zeros_like(xa)
    acc1 = jnp.zeros_like(xa)
    for k in range(n_lags):
        sh = _shift_left(xa, k)
        acc0 = acc0 + sh * filt[:, k:k + 1]
        acc1 = acc1 + sh * filt[:, n_lags + k:n_lags + k + 1]
    o_ref[...] = jnp.concatenate([acc0[:, :win], acc1[:, :win]], axis=1)[None]


def _pallas_fused(xa4, stats, w_enc, b_enc, gamma, beta, w_bn, b_bn,
                  w1, b1, w2, b2, slopes, w_fg, b_fg,
                  *, tm, n_layer, n_spk, n_lags, win, cw, enc_dim, half):
    nchan, nw, _ = xa4.shape
    feat = w_bn.shape[1]
    hid = w1.shape[2]
    ntile = nw // tm
    kern = functools.partial(_fused_kernel, n_layer=n_layer, n_spk=n_spk,
                             n_lags=n_lags, win=win, cw=cw, enc_dim=enc_dim,
                             half=half)
    return pl.pallas_call(
        kern,
        out_shape=jax.ShapeDtypeStruct((nchan, nw, n_spk * win), jnp.float32),
        grid_spec=pltpu.PrefetchScalarGridSpec(
            num_scalar_prefetch=0,
            grid=(nchan, ntile),
            in_specs=[
                pl.BlockSpec((1, tm, LANE), lambda c, j: (c, j, 0)),
                pl.BlockSpec((1, 1, LANE), lambda c, j: (c, 0, 0)),
                pl.BlockSpec((LANE, enc_dim), lambda c, j: (0, 0)),
                pl.BlockSpec((1, enc_dim), lambda c, j: (0, 0)),
                pl.BlockSpec((1, enc_dim), lambda c, j: (0, 0)),
                pl.BlockSpec((1, enc_dim), lambda c, j: (0, 0)),
                pl.BlockSpec((LANE, feat), lambda c, j: (0, 0)),
                pl.BlockSpec((1, feat), lambda c, j: (0, 0)),
                pl.BlockSpec((n_layer, feat, hid), lambda c, j: (0, 0, 0)),
                pl.BlockSpec((n_layer, 1, hid), lambda c, j: (0, 0, 0)),
                pl.BlockSpec((n_layer, hid, feat), lambda c, j: (0, 0, 0)),
                pl.BlockSpec((n_layer, 1, feat), lambda c, j: (0, 0, 0)),
                pl.BlockSpec((n_layer, 1, hid), lambda c, j: (0, 0, 0)),
                pl.BlockSpec((feat, 2 * half), lambda c, j: (0, 0)),
                pl.BlockSpec((1, 2 * half), lambda c, j: (0, 0)),
            ],
            out_specs=pl.BlockSpec((1, tm, n_spk * win), lambda c, j: (c, j, 0)),
        ),
        compiler_params=pltpu.CompilerParams(
            dimension_semantics=("parallel", "parallel")),
    )(xa4, stats, w_enc, b_enc, gamma, beta, w_bn, b_bn,
      w1, b1, w2, b2, slopes, w_fg, b_fg)


def kernel(mix, W_enc, b_enc, ln_gamma, ln_beta, W_bn, b_bn, W1, b1, W2, b2,
           prelu, W_f, b_f, W_g, b_g):
    b, nmic, t = mix.shape
    win = WIN
    context = CONTEXT
    stride = win // 2
    cw = 2 * context + win
    ntaps = 2 * context + 1
    enc_dim = W_enc.shape[1]
    feat = W_bn.shape[1]
    hid = W1.shape[2]
    nlayer = W1.shape[0]
    nspk = W_f.shape[1] // ntaps
    half = _round_up(nspk * ntaps, LANE)
    nchan = b * nmic

    rest = (win - (stride + t % win) % win) % win
    nsample = t + rest + 2 * stride
    n_win = 2 * nsample // win - 1

    tm = 512
    nw = _round_up(n_win, tm)

    mix32 = mix.astype(jnp.float32)
    x = jnp.pad(mix32, ((0, 0), (0, 0), (stride, rest + stride)))
    total = nw * stride + cw
    xc = jnp.pad(x, ((0, 0), (0, 0), (context, total - nsample - context)))
    nchunk = total // stride
    chunks = xc.reshape(nchan, nchunk, stride)
    ctx4 = jnp.concatenate(
        [chunks[:, m:m + nw, :] for m in range(cw // stride)], axis=2)
    ref4 = ctx4.reshape(b, nmic, nw, cw)[:, 0:1, :, context:context + win]
    ref4 = jnp.broadcast_to(ref4, (b, nmic, nw, win)).reshape(nchan, nw, win)
    xa4 = jnp.concatenate(
        [ctx4, ref4, jnp.zeros((nchan, nw, LANE - cw - win), jnp.float32)],
        axis=2)

    w_enc = jnp.zeros((LANE, enc_dim), jnp.float32).at[:cw].set(W_enc)
    w_enc = w_enc.astype(jnp.bfloat16)
    gamma = ln_gamma.reshape(1, enc_dim)
    beta = ln_beta.reshape(1, enc_dim)
    w_bn = jnp.zeros((LANE, feat), jnp.float32).at[:enc_dim + ntaps].set(W_bn)
    w_bn = w_bn.astype(jnp.bfloat16)
    wpad = half - nspk * ntaps
    w_fg = jnp.concatenate([jnp.pad(W_f, ((0, 0), (0, wpad))),
                            jnp.pad(W_g, ((0, 0), (0, wpad)))],
                           axis=1).astype(jnp.bfloat16)
    b_fg = jnp.concatenate([jnp.pad(b_f, ((0, 0), (0, wpad))),
                            jnp.pad(b_g, ((0, 0), (0, wpad)))], axis=1)
    slopes = prelu[:, None, None] * jnp.ones((1, 1, hid), jnp.float32)

    parts = _pallas_enc_stats(xa4, w_enc, b_enc, tm=tm, n_win=n_win)
    sums = parts.reshape(nchan, -1, 2 * enc_dim).sum(axis=1)
    n_el = n_win * enc_dim
    mean = sums[:, :enc_dim].sum(axis=1) / n_el
    var = sums[:, enc_dim:].sum(axis=1) / n_el - mean * mean
    rstd = jax.lax.rsqrt(var + 1e-8)
    stats = jnp.zeros((nchan, 1, LANE), jnp.float32)
    stats = stats.at[:, 0, 0].set(mean).at[:, 0, 1].set(rstd)

    out4 = _pallas_fused(
        xa4, stats, w_enc, b_enc, gamma, beta, w_bn, b_bn,
        W1.astype(jnp.bfloat16), b1, W2.astype(jnp.bfloat16), b2,
        slopes, w_fg, b_fg,
        tm=tm, n_layer=nlayer, n_spk=nspk, n_lags=ntaps, win=win, cw=cw,
        enc_dim=enc_dim, half=half)

    bf = out4[:, :n_win, :].reshape(b, nmic, n_win, nspk, win).sum(axis=1)
    bf = jnp.transpose(bf, (0, 2, 1, 3)).reshape(b * nspk, n_win, win)
    first = bf[:, :, :stride]
    second = bf[:, :, stride:]
    sig = (jnp.pad(first, ((0, 0), (0, 1), (0, 0)))
           + jnp.pad(second, ((0, 0), (1, 0), (0, 0))))
    sig = sig.reshape(b * nspk, nsample)[:, stride:stride + t]
    return sig.reshape(b, nspk, t)[:, :, None, :]


# speedup vs baseline: 1.4323x; 1.4110x over previous
import functools

import jax
import jax.numpy as jnp
from jax.experimental import pallas as pl
from jax.experimental.pallas import tpu as pltpu

LANE = 128
WIN = 16
CONTEXT = 16


def _round_up(x, m):
    return (x + m - 1) // m * m


def _shift_left(x, n):
    if n == 0:
        return x
    pad = jnp.zeros(x.shape[:-1] + (n,), x.dtype)
    return jnp.concatenate([x[:, n:], pad], axis=1)


def _enc_stats_kernel(xa_ref, w_enc_ref, b_enc_ref, o_ref, *, tm, n_win):
    xa = xa_ref[0]
    enc = jnp.dot(xa.astype(jnp.bfloat16), w_enc_ref[...],
                  preferred_element_type=jnp.float32) + b_enc_ref[...]
    j = pl.program_id(1)
    row = jax.lax.broadcasted_iota(jnp.int32, (tm, 1), 0)
    mask = (row + j * tm) < n_win
    encm = jnp.where(mask, enc, 0.0)
    s = jnp.sum(encm, axis=0, keepdims=True)
    sq = jnp.sum(encm * encm, axis=0, keepdims=True)
    o_ref[...] = jnp.concatenate([s, sq], axis=1)[None, None]


def _pallas_enc_stats(xa4, w_enc, b_enc, *, tm, n_win):
    nchan, nw, _ = xa4.shape
    enc = w_enc.shape[1]
    ntile = nw // tm
    kern = functools.partial(_enc_stats_kernel, tm=tm, n_win=n_win)
    return pl.pallas_call(
        kern,
        out_shape=jax.ShapeDtypeStruct((nchan, ntile, 1, 2 * enc), jnp.float32),
        grid_spec=pltpu.PrefetchScalarGridSpec(
            num_scalar_prefetch=0,
            grid=(nchan, ntile),
            in_specs=[
                pl.BlockSpec((1, tm, LANE), lambda c, j: (c, j, 0)),
                pl.BlockSpec((LANE, enc), lambda c, j: (0, 0)),
                pl.BlockSpec((1, enc), lambda c, j: (0, 0)),
            ],
            out_specs=pl.BlockSpec((1, 1, 1, 2 * enc), lambda c, j: (c, j, 0, 0)),
        ),
        compiler_params=pltpu.CompilerParams(
            dimension_semantics=("parallel", "parallel")),
    )(xa4, w_enc, b_enc)


def _fused_kernel(xa_ref, stats_ref, w_enc_ref, b_enc_ref, gamma_ref, beta_ref,
                  w_bn_ref, b_bn_ref, w1_ref, b1_ref, w2_ref, b2_ref,
                  slope_ref, w_fg_ref, b_fg_ref, mred_ref, o_ref,
                  *, n_layer, n_spk, n_lags, win, cw, enc_dim, half):
    xa = xa_ref[0]
    tm = xa.shape[0]

    enc = jnp.dot(xa.astype(jnp.bfloat16), w_enc_ref[...],
                  preferred_element_type=jnp.float32) + b_enc_ref[...]

    num0 = jnp.zeros_like(xa)
    num1 = jnp.zeros_like(xa)
    seg0 = jnp.zeros_like(xa)
    seg1 = jnp.zeros_like(xa)
    for t in range(win):
        sh = _shift_left(xa, t)
        r_t = xa[:, cw + t:cw + t + 1]
        if t % 2 == 0:
            num0 = num0 + sh * r_t
            seg0 = seg0 + sh * sh
        else:
            num1 = num1 + sh * r_t
            seg1 = seg1 + sh * sh
    num = num0 + num1
    seg_sq = seg0 + seg1
    ref = xa[:, cw:cw + win]
    ref_sq = jnp.sum(ref * ref, axis=1, keepdims=True)
    den = jnp.sqrt(seg_sq * ref_sq) + 1e-8
    ncc = num * pl.reciprocal(den, approx=True)

    st = stats_ref[0]
    mean = st[:, 0:1]
    rstd = st[:, 1:2]
    enc_ln = (enc - mean) * rstd * gamma_ref[...] + beta_ref[...]

    pad = LANE - enc_dim - n_lags
    x_in = jnp.concatenate(
        [enc_ln, ncc[:, :n_lags], jnp.zeros((tm, pad), jnp.float32)], axis=1)
    h = jnp.dot(x_in.astype(jnp.bfloat16), w_bn_ref[...],
                preferred_element_type=jnp.float32) + b_bn_ref[...]

    for l in range(n_layer):
        y = jnp.dot(h.astype(jnp.bfloat16), w1_ref[l],
                    preferred_element_type=jnp.float32) + b1_ref[l]
        y = jnp.where(y >= 0.0, y, slope_ref[l] * y)
        h = h + jnp.dot(y.astype(jnp.bfloat16), w2_ref[l],
                        preferred_element_type=jnp.float32) + b2_ref[l]

    fg = jnp.dot(h.astype(jnp.bfloat16), w_fg_ref[...],
                 preferred_element_type=jnp.float32) + b_fg_ref[...]
    filt = jnp.tanh(fg[:, :half]) * jax.nn.sigmoid(fg[:, half:])

    dup = jnp.concatenate([xa[:, :64], xa[:, :64]], axis=1)
    acc = jnp.zeros((tm, n_spk * win), jnp.float32)
    for t in range(win):
        sh = _shift_left(dup, t)
        tmp = (sh * filt).astype(jnp.bfloat16)
        acc = acc + jnp.dot(tmp, mred_ref[t],
                            preferred_element_type=jnp.float32)
    o_ref[...] = acc[None]


def _pallas_fused(xa4, stats, w_enc, b_enc, gamma, beta, w_bn, b_bn,
                  w1, b1, w2, b2, slopes, w_fg, b_fg, mred,
                  *, tm, n_layer, n_spk, n_lags, win, cw, enc_dim, half):
    nchan, nw, _ = xa4.shape
    feat = w_bn.shape[1]
    hid = w1.shape[2]
    ntile = nw // tm
    kern = functools.partial(_fused_kernel, n_layer=n_layer, n_spk=n_spk,
                             n_lags=n_lags, win=win, cw=cw, enc_dim=enc_dim,
                             half=half)
    return pl.pallas_call(
        kern,
        out_shape=jax.ShapeDtypeStruct((nchan, nw, n_spk * win), jnp.float32),
        grid_spec=pltpu.PrefetchScalarGridSpec(
            num_scalar_prefetch=0,
            grid=(nchan, ntile),
            in_specs=[
                pl.BlockSpec((1, tm, LANE), lambda c, j: (c, j, 0)),
                pl.BlockSpec((1, 1, LANE), lambda c, j: (c, 0, 0)),
                pl.BlockSpec((LANE, enc_dim), lambda c, j: (0, 0)),
                pl.BlockSpec((1, enc_dim), lambda c, j: (0, 0)),
                pl.BlockSpec((1, enc_dim), lambda c, j: (0, 0)),
                pl.BlockSpec((1, enc_dim), lambda c, j: (0, 0)),
                pl.BlockSpec((LANE, feat), lambda c, j: (0, 0)),
                pl.BlockSpec((1, feat), lambda c, j: (0, 0)),
                pl.BlockSpec((n_layer, feat, hid), lambda c, j: (0, 0, 0)),
                pl.BlockSpec((n_layer, 1, hid), lambda c, j: (0, 0, 0)),
                pl.BlockSpec((n_layer, hid, feat), lambda c, j: (0, 0, 0)),
                pl.BlockSpec((n_layer, 1, feat), lambda c, j: (0, 0, 0)),
                pl.BlockSpec((n_layer, 1, hid), lambda c, j: (0, 0, 0)),
                pl.BlockSpec((feat, 2 * half), lambda c, j: (0, 0)),
                pl.BlockSpec((1, 2 * half), lambda c, j: (0, 0)),
                pl.BlockSpec((win, LANE, n_spk * win), lambda c, j: (0, 0, 0)),
            ],
            out_specs=pl.BlockSpec((1, tm, n_spk * win), lambda c, j: (c, j, 0)),
        ),
        compiler_params=pltpu.CompilerParams(
            dimension_semantics=("parallel", "parallel")),
    )(xa4, stats, w_enc, b_enc, gamma, beta, w_bn, b_bn,
      w1, b1, w2, b2, slopes, w_fg, b_fg, mred)


def kernel(mix, W_enc, b_enc, ln_gamma, ln_beta, W_bn, b_bn, W1, b1, W2, b2,
           prelu, W_f, b_f, W_g, b_g):
    b, nmic, t = mix.shape
    win = WIN
    context = CONTEXT
    stride = win // 2
    cw = 2 * context + win
    ntaps = 2 * context + 1
    enc_dim = W_enc.shape[1]
    feat = W_bn.shape[1]
    hid = W1.shape[2]
    nlayer = W1.shape[0]
    nspk = W_f.shape[1] // ntaps
    half = _round_up(nspk * ntaps, LANE)
    nchan = b * nmic

    rest = (win - (stride + t % win) % win) % win
    nsample = t + rest + 2 * stride
    n_win = 2 * nsample // win - 1

    tm_stats = 512
    tm = 512
    nw = _round_up(n_win, tm_stats)

    mix32 = mix.astype(jnp.float32)
    x = jnp.pad(mix32, ((0, 0), (0, 0), (stride, rest + stride)))
    total = nw * stride + cw
    xc = jnp.pad(x, ((0, 0), (0, 0), (context, total - nsample - context)))
    nchunk = total // stride
    chunks = xc.reshape(nchan, nchunk, stride)
    ctx4 = jnp.concatenate(
        [chunks[:, m:m + nw, :] for m in range(cw // stride)], axis=2)
    ref4 = ctx4.reshape(b, nmic, nw, cw)[:, 0:1, :, context:context + win]
    ref4 = jnp.broadcast_to(ref4, (b, nmic, nw, win)).reshape(nchan, nw, win)
    xa4 = jnp.concatenate(
        [ctx4, ref4, jnp.zeros((nchan, nw, LANE - cw - win), jnp.float32)],
        axis=2)

    w_enc = jnp.zeros((LANE, enc_dim), jnp.float32).at[:cw].set(W_enc)
    w_enc = w_enc.astype(jnp.bfloat16)
    gamma = ln_gamma.reshape(1, enc_dim)
    beta = ln_beta.reshape(1, enc_dim)
    w_bn = jnp.zeros((LANE, feat), jnp.float32).at[:enc_dim + ntaps].set(W_bn)
    w_bn = w_bn.astype(jnp.bfloat16)
    def _perm(w):
        out = jnp.zeros((w.shape[0], half), w.dtype)
        out = out.at[:, :ntaps].set(w[:, :ntaps])
        return out.at[:, 64:64 + ntaps].set(w[:, ntaps:2 * ntaps])
    w_fg = jnp.concatenate([_perm(W_f), _perm(W_g)], axis=1).astype(jnp.bfloat16)
    b_fg = jnp.concatenate([_perm(b_f), _perm(b_g)], axis=1)
    mred = jnp.zeros((win, LANE, nspk * win), jnp.float32)
    for tt in range(win):
        mred = mred.at[tt, :ntaps, tt].set(1.0)
        mred = mred.at[tt, 64:64 + ntaps, win + tt].set(1.0)
    mred = mred.astype(jnp.bfloat16)
    slopes = prelu[:, None, None] * jnp.ones((1, 1, hid), jnp.float32)

    parts = _pallas_enc_stats(xa4, w_enc, b_enc, tm=tm_stats, n_win=n_win)
    sums = parts.reshape(nchan, -1, 2 * enc_dim).sum(axis=1)
    n_el = n_win * enc_dim
    mean = sums[:, :enc_dim].sum(axis=1) / n_el
    var = sums[:, enc_dim:].sum(axis=1) / n_el - mean * mean
    rstd = jax.lax.rsqrt(var + 1e-8)
    stats = jnp.zeros((nchan, 1, LANE), jnp.float32)
    stats = stats.at[:, 0, 0].set(mean).at[:, 0, 1].set(rstd)

    out4 = _pallas_fused(
        xa4, stats, w_enc, b_enc, gamma, beta, w_bn, b_bn,
        W1.astype(jnp.bfloat16), b1, W2.astype(jnp.bfloat16), b2,
        slopes, w_fg, b_fg, mred,
        tm=tm, n_layer=nlayer, n_spk=nspk, n_lags=ntaps, win=win, cw=cw,
        enc_dim=enc_dim, half=half)

    bf = out4[:, :n_win, :].reshape(b, nmic, n_win, nspk, win).sum(axis=1)
    bf = jnp.transpose(bf, (0, 2, 1, 3)).reshape(b * nspk, n_win, win)
    first = bf[:, :, :stride]
    second = bf[:, :, stride:]
    sig = (jnp.pad(first, ((0, 0), (0, 1), (0, 0)))
           + jnp.pad(second, ((0, 0), (1, 0), (0, 0))))
    sig = sig.reshape(b * nspk, nsample)[:, stride:stride + t]
    return sig.reshape(b, nspk, t)[:, :, None, :]
